# Initial kernel scaffold; baseline (speedup 1.0000x reference)
import jax
import jax.numpy as jnp
from jax.experimental import pallas as pl


def kernel(x, edge_index, edge_attr, batch, params):
    raise NotImplementedError("write your pallas kernel here")


# speedup vs baseline: 2.4639x; 2.4639x over previous
import functools

import jax
import jax.numpy as jnp
import numpy as np
from jax import lax
from jax.experimental import pallas as pl
from jax.experimental.pallas import tpu as pltpu
from jax.experimental.pallas import tpu_sc as plsc

N = 10000
E = 320000
D_EDGE = 16
H = 128
G = 64
OUT = 128
BN_EPS = 1e-5

NW = 32
CHUNK = 128
CHUNKS_PER_W = 79
E_PAD = NW * CHUNKS_PER_W * CHUNK
N_AGG = 10240
ROWS_PER_TILE = N_AGG // 16


def _sc_propagate(h, ee, src_p, dst_p):
    mesh = plsc.VectorSubcoreMesh(core_axis_name="c", subcore_axis_name="s")

    @functools.partial(
        pl.kernel,
        mesh=mesh,
        out_type=jax.ShapeDtypeStruct((2, N_AGG, H), jnp.float32),
        scratch_types=[
            pltpu.VMEM((CHUNK,), jnp.int32),
            pltpu.VMEM((CHUNK,), jnp.int32),
            pltpu.VMEM((CHUNK, H), jnp.float32),
            pltpu.VMEM((CHUNK, H), jnp.float32),
            pltpu.VMEM_SHARED((N_AGG, H), jnp.float32),
            pltpu.SemaphoreType.DMA,
        ],
    )
    def body(h_hbm, ee_hbm, src_hbm, dst_hbm, out_hbm,
             src_v, dst_v, rows_v, ee_v, agg_sh, sem):
        cid = lax.axis_index("c")
        sid = lax.axis_index("s")
        wid = cid * 16 + sid

        def zbody(i, carry):
            for j in range(8):
                ee_v[i, pl.ds(j * 16, 16)] = jnp.zeros((16,), jnp.float32)
            return carry
        lax.fori_loop(0, CHUNK, zbody, 0)
        for c in range(ROWS_PER_TILE // CHUNK):
            pltpu.sync_copy(
                ee_v, agg_sh.at[pl.ds(sid * ROWS_PER_TILE + c * CHUNK, CHUNK)])
        plsc.subcore_barrier()

        def chunk_body(g, carry):
            base = wid * (CHUNKS_PER_W * CHUNK) + g * CHUNK
            pltpu.sync_copy(src_hbm.at[pl.ds(base, CHUNK)], src_v)
            pltpu.sync_copy(dst_hbm.at[pl.ds(base, CHUNK)], dst_v)
            gat = pltpu.async_copy(h_hbm.at[src_v], rows_v, sem)
            pltpu.sync_copy(ee_hbm.at[pl.ds(base, CHUNK)], ee_v)
            gat.wait()

            def rbody(i, c2):
                for j in range(8):
                    s = pl.ds(j * 16, 16)
                    rows_v[i, s] = jnp.maximum(rows_v[i, s] + ee_v[i, s], 0.0)
                return c2
            lax.fori_loop(0, CHUNK, rbody, 0)
            pltpu.sync_copy(rows_v, agg_sh.at[dst_v], add=True)
            return carry
        lax.fori_loop(0, CHUNKS_PER_W, chunk_body, 0)
        plsc.subcore_barrier()

        for c in range(ROWS_PER_TILE // CHUNK):
            r0 = sid * ROWS_PER_TILE + c * CHUNK
            pltpu.sync_copy(agg_sh.at[pl.ds(r0, CHUNK)], ee_v)
            pltpu.sync_copy(ee_v, out_hbm.at[cid, pl.ds(r0, CHUNK)])

    return body(h, ee, src_p, dst_p)


def _edge_mlp(ea_p, wl):
    BE = 2048
    grid = E_PAD // BE

    def body(ea_ref, *refs):
        ea = ea_ref[...]
        for l in range(3):
            w1 = refs[4 * l][...]
            b1 = refs[4 * l + 1][...]
            w2 = refs[4 * l + 2][...]
            b2 = refs[4 * l + 3][...]
            t = jnp.maximum(
                jax.lax.dot_general(ea, w1, (((1,), (0,)), ((), ())),
                                    preferred_element_type=jnp.float32) + b1,
                0.0)
            refs[12 + l][...] = jax.lax.dot_general(
                t, w2, (((1,), (0,)), ((), ())),
                preferred_element_type=jnp.float32) + b2

    wspec = pl.BlockSpec((D_EDGE, H), lambda i: (0, 0))
    hspec = pl.BlockSpec((H, H), lambda i: (0, 0))
    bspec = pl.BlockSpec((1, H), lambda i: (0, 0))
    in_specs = [pl.BlockSpec((BE, D_EDGE), lambda i: (i, 0))]
    args = [ea_p]
    for p in wl:
        in_specs += [wspec, bspec, hspec, bspec]
        args += [p['be1_w'], p['be1_b'].reshape(1, H),
                 p['be2_w'], p['be2_b'].reshape(1, H)]

    return pl.pallas_call(
        body,
        grid=(grid,),
        in_specs=in_specs,
        out_specs=[pl.BlockSpec((BE, H), lambda i: (i, 0))] * 3,
        out_shape=[jax.ShapeDtypeStruct((E_PAD, H), jnp.float32)] * 3,
    )(*args)


_INV_BN = float(1.0 / np.sqrt(1.0 + BN_EPS))


def _node_mlp(h, a0, a1, p):
    BNR = 2000
    grid = N // BNR
    eps1 = (1.0 + p['eps']).reshape(1, 1) * jnp.ones((1, H), jnp.float32)

    def body(h_ref, a0_ref, a1_ref, e_ref, w1_ref, b1_ref, w2_ref, b2_ref,
             g_ref, bb_ref, o_ref):
        z = h_ref[...] * e_ref[...] + a0_ref[...] + a1_ref[...]
        t = jnp.maximum(
            jax.lax.dot_general(z, w1_ref[...], (((1,), (0,)), ((), ())),
                                preferred_element_type=jnp.float32)
            + b1_ref[...], 0.0)
        y = jax.lax.dot_general(t, w2_ref[...], (((1,), (0,)), ((), ())),
                                preferred_element_type=jnp.float32) + b2_ref[...]
        y = jnp.maximum(y, 0.0)
        o_ref[...] = y * (g_ref[...] * _INV_BN) + bb_ref[...]

    rows = pl.BlockSpec((BNR, H), lambda i: (i, 0))
    full = pl.BlockSpec((H, H), lambda i: (0, 0))
    vec = pl.BlockSpec((1, H), lambda i: (0, 0))
    return pl.pallas_call(
        body,
        grid=(grid,),
        in_specs=[rows, rows, rows, vec, full, vec, full, vec, vec, vec],
        out_specs=rows,
        out_shape=jax.ShapeDtypeStruct((N, H), jnp.float32),
    )(h, a0, a1, eps1, p['m1_w'], p['m1_b'].reshape(1, H),
      p['m2_w'], p['m2_b'].reshape(1, H),
      p['bn_g'].reshape(1, H), p['bn_b'].reshape(1, H))


def _pool_fc(h1, h2, h3, batch3, f1a, f1b, f1c, fc1_b, fc4_w, fc4_b):
    BR = 1000
    grid = N // BR

    def body(b_ref, h1_ref, h2_ref, h3_ref, f1a_ref, f1b_ref, f1c_ref,
             fb1_ref, f4_ref, fb4_ref, o_ref, s1, s2, s3, cnt):
        i = pl.program_id(0)

        @pl.when(i == 0)
        def _():
            s1[...] = jnp.zeros((G, H), jnp.float32)
            s2[...] = jnp.zeros((G, H), jnp.float32)
            s3[...] = jnp.zeros((G, H), jnp.float32)
            cnt[...] = jnp.zeros((G, H), jnp.float32)

        b = b_ref[0]
        iota = lax.broadcasted_iota(jnp.int32, (G, BR), 0)
        oh = (iota == b).astype(jnp.float32)

        def mm(a, c):
            return jax.lax.dot_general(a, c, (((1,), (0,)), ((), ())),
                                       preferred_element_type=jnp.float32)

        s1[...] += mm(oh, h1_ref[...])
        s2[...] += mm(oh, h2_ref[...])
        s3[...] += mm(oh, h3_ref[...])
        cnt[...] += mm(oh, jnp.ones((BR, H), jnp.float32))

        @pl.when(i == grid - 1)
        def _():
            inv = 1.0 / jnp.maximum(cnt[:, 0:1], 1.0)
            m1 = s1[...] * inv
            m2 = s2[...] * inv
            m3 = s3[...] * inv
            hf = jnp.maximum(
                mm(m1, f1a_ref[...]) + mm(m2, f1b_ref[...])
                + mm(m3, f1c_ref[...]) + fb1_ref[...], 0.0)
            o_ref[...] = mm(hf, f4_ref[...]) + fb4_ref[...]

    rows = pl.BlockSpec((BR, H), lambda i: (i, 0))
    full = pl.BlockSpec((H, H), lambda i: (0, 0))
    vec = pl.BlockSpec((1, H), lambda i: (0, 0))
    bspec = pl.BlockSpec((1, 1, BR), lambda i: (i, 0, 0))
    return pl.pallas_call(
        body,
        grid=(grid,),
        in_specs=[bspec, rows, rows, rows, full, full, full, vec, full, vec],
        out_specs=pl.BlockSpec((G, OUT), lambda i: (0, 0)),
        out_shape=jax.ShapeDtypeStruct((G, OUT), jnp.float32),
        scratch_shapes=[pltpu.VMEM((G, H), jnp.float32)] * 4,
    )(batch3, h1, h2, h3, f1a, f1b, f1c, fc1_b.reshape(1, H),
      fc4_w, fc4_b.reshape(1, OUT))


def kernel(x, edge_index, edge_attr, batch, params):
    src = edge_index[0]
    dst = edge_index[1]
    pad = E_PAD - E
    src_p = jnp.concatenate([src, jnp.zeros((pad,), jnp.int32)])
    dst_p = jnp.concatenate([dst, jnp.full((pad,), N, jnp.int32)])
    ea_p = jnp.concatenate([edge_attr, jnp.zeros((pad, D_EDGE), jnp.float32)])

    ee = _edge_mlp(ea_p, params['layers'])

    hs = []
    h = x
    for l, p in enumerate(params['layers']):
        aggs = _sc_propagate(h, ee[l], src_p, dst_p)
        h = _node_mlp(h, aggs[0, :N], aggs[1, :N], p)
        hs.append(h)

    batch3 = batch.reshape(N // 1000, 1, 1000)
    f1 = params['fc1_w']
    return _pool_fc(hs[0], hs[1], hs[2], batch3,
                    f1[0:H], f1[H:2 * H], f1[2 * H:3 * H],
                    params['fc1_b'], params['fc4_w'], params['fc4_b'])
